# baseline (speedup 1.0000x reference)
import functools

import jax
import jax.numpy as jnp
from jax import lax
from jax.experimental import pallas as pl
from jax.experimental.pallas import tpu as pltpu
from jax.experimental.pallas import tpu_sc as plsc

_H = 8
_DH = 16
_NC = 2
_NS = 16
_L = 16


def _proj_body(nodes_ref, wsrc_ref, bsrc_ref, wdst_ref, bdst_ref,
               wval_ref, bval_ref, xsrc_ref, xdst_ref, vals_ref):
    x = nodes_ref[...]
    xsrc_ref[...] = jnp.dot(x, wsrc_ref[...],
                            preferred_element_type=jnp.float32) + bsrc_ref[...]
    xdst_ref[...] = jnp.dot(x, wdst_ref[...],
                            preferred_element_type=jnp.float32) + bdst_ref[...]
    vals_ref[...] = jnp.dot(x, wval_ref[...],
                            preferred_element_type=jnp.float32) + bval_ref[...]


def _node_proj(nodes, w_src, b_src, w_dst, b_dst, w_val, b_val, block=1000):
    n, d = nodes.shape
    grid = n // block
    wspec = pl.BlockSpec((d, d), lambda i: (0, 0))
    bspec = pl.BlockSpec((1, d), lambda i: (0, 0))
    xspec = pl.BlockSpec((block, d), lambda i: (i, 0))
    out = jax.ShapeDtypeStruct((n, d), jnp.float32)
    return pl.pallas_call(
        _proj_body,
        grid=(grid,),
        in_specs=[xspec, wspec, bspec, wspec, bspec, wspec, bspec],
        out_specs=[xspec, xspec, xspec],
        out_shape=[out, out, out],
    )(nodes, w_src, b_src[None, :], w_dst, b_dst[None, :], w_val, b_val[None, :])


def _gather_body(epw, bb, psrc, pdst, isrc, idst, s_out,
                 iv1a, iv2a, g1a, g2a, iv1b, iv2b, g1b, g2b,
                 sia, sib, sga, sgb, ssa, ssb):
    cid = lax.axis_index("c")
    sid = lax.axis_index("s")
    wid = sid * _NC + cid
    base = wid * epw
    nb = epw // bb
    bufs = ((iv1a, iv2a, g1a, g2a, sia, sga, ssa),
            (iv1b, iv2b, g1b, g2b, sib, sgb, ssb))

    def fire_idx(j, b):
        iv1, iv2, _, _, si, _, _ = bufs[b]
        off = base + j * bb
        pltpu.async_copy(isrc.at[pl.ds(off, bb)], iv1, si)
        pltpu.async_copy(idst.at[pl.ds(off, bb)], iv2, si)

    def wait_idx(b):
        iv1, iv2, _, _, si, _, _ = bufs[b]
        pltpu.make_async_copy(isrc.at[pl.ds(base, bb)], iv1, si).wait()
        pltpu.make_async_copy(isrc.at[pl.ds(base, bb)], iv2, si).wait()

    def fire_gather(b):
        iv1, iv2, g1, g2, _, sg, _ = bufs[b]
        pltpu.async_copy(psrc.at[iv1], g1, sg)
        pltpu.async_copy(pdst.at[iv2], g2, sg)

    def wait_gather(b):
        iv1, iv2, g1, g2, _, sg, _ = bufs[b]
        pltpu.make_async_copy(psrc.at[iv1], g1, sg).wait()
        pltpu.make_async_copy(pdst.at[iv2], g2, sg).wait()

    def fire_store(j, b):
        _, _, g1, _, _, _, ss = bufs[b]
        off = base + j * bb
        pltpu.async_copy(g1, s_out.at[pl.ds(off, bb)], ss)

    def wait_store(b):
        _, _, g1, _, _, _, ss = bufs[b]
        pltpu.make_async_copy(g1, s_out.at[pl.ds(base, bb)], ss).wait()

    def compute(b):
        _, _, g1, g2, _, _, _ = bufs[b]

        def erow(e, c2):
            for h in range(_H):
                sl = pl.ds(h * _L, _L)
                g1[e, sl] = g1[e, sl] + g2[e, sl]
            return c2

        lax.fori_loop(0, bb, erow, 0)

    fire_idx(0, 0)
    fire_idx(1, 1)
    wait_idx(0)
    fire_gather(0)

    def pair(j2, carry):
        for b in range(2):
            j = j2 * 2 + b
            wait_gather(b)
            compute(b)
            fire_store(j, b)
            wait_idx(1 - b)

            @pl.when(j2 * 2 + b >= 1)
            def _():
                wait_store(1 - b)

            fire_gather(1 - b)
            @pl.when(j + 2 <= nb - 1)
            def _():
                fire_idx(j + 2, b)
        return carry

    lax.fori_loop(0, (nb - 1) // 2, pair, 0)
    bl = (nb - 1) % 2
    wait_gather(bl)
    compute(bl)
    fire_store(nb - 1, bl)
    wait_store(1 - bl)
    wait_store(bl)


def _edge_gather_sum(x_src, x_dst, isrc, idst):
    n, d = x_src.shape
    e = isrc.shape[0]
    epw = e // (_NC * _NS)
    bb = 80
    mesh = plsc.VectorSubcoreMesh(core_axis_name="c", subcore_axis_name="s", num_cores=_NC, num_subcores=_NS)
    kern = pl.kernel(
        functools.partial(_gather_body, epw, bb),
        out_type=jax.ShapeDtypeStruct((e, d), jnp.float32),
        mesh=mesh,
        scratch_types=[
            pltpu.VMEM((bb,), jnp.int32),
            pltpu.VMEM((bb,), jnp.int32),
            pltpu.VMEM((bb, d), jnp.float32),
            pltpu.VMEM((bb, d), jnp.float32),
            pltpu.VMEM((bb,), jnp.int32),
            pltpu.VMEM((bb,), jnp.int32),
            pltpu.VMEM((bb, d), jnp.float32),
            pltpu.VMEM((bb, d), jnp.float32),
            pltpu.SemaphoreType.DMA,
            pltpu.SemaphoreType.DMA,
            pltpu.SemaphoreType.DMA,
            pltpu.SemaphoreType.DMA,
            pltpu.SemaphoreType.DMA,
            pltpu.SemaphoreType.DMA,
        ],
    )
    return kern(x_src, x_dst, isrc, idst)


def _edge_pre_body(edges_ref, spe_ref, wedge_ref, bedge_ref, ep_ref):
    ep_ref[...] = jnp.dot(edges_ref[...], wedge_ref[...],
                          preferred_element_type=jnp.float32) \
        + bedge_ref[...] + spe_ref[...]


def _edge_pre(edges, spe, w_edge, b_edge, block=512):
    e, d = edges.shape
    espec = pl.BlockSpec((block, d), lambda i: (i, 0))
    wspec = pl.BlockSpec((d, d), lambda i: (0, 0))
    bspec = pl.BlockSpec((1, d), lambda i: (0, 0))
    return pl.pallas_call(
        _edge_pre_body,
        grid=(e // block,),
        in_specs=[espec, espec, wspec, bspec],
        out_specs=espec,
        out_shape=jax.ShapeDtypeStruct((e, d), jnp.float32),
    )(edges, spe, w_edge, b_edge[None, :])


def _edge_body(ep_ref, s_ref, weout_ref, beout_ref, amat_ref, ab_ref,
               ne_ref, w_ref):
    z = ep_ref[...] + s_ref[...]
    t = 0.5 * z * (1.0 + lax.erf(z * 0.7071067811865476))
    ne_ref[...] = jnp.dot(t, weout_ref[...],
                          preferred_element_type=jnp.float32) + beout_ref[...]
    logits = jnp.dot(t, amat_ref[...],
                     preferred_element_type=jnp.float32) + ab_ref[...]
    w_ref[...] = jnp.exp(logits).T


def _edge_stage(ep, s, w_eout, b_eout, a_w, a_b, block=512):
    e, d = ep.shape
    grid = e // block
    amat = jnp.zeros((d, _H), jnp.float32)
    rows = jnp.arange(d)
    amat = amat.at[rows, rows // _DH].set(a_w.reshape(d))
    espec = pl.BlockSpec((block, d), lambda i: (i, 0))
    wspec = pl.BlockSpec((d, d), lambda i: (0, 0))
    bspec = pl.BlockSpec((1, d), lambda i: (0, 0))
    aspec = pl.BlockSpec((d, _H), lambda i: (0, 0))
    abspec = pl.BlockSpec((1, _H), lambda i: (0, 0))
    wout_spec = pl.BlockSpec((_H, block), lambda i: (0, i))
    return pl.pallas_call(
        _edge_body,
        grid=(grid,),
        in_specs=[espec, espec, wspec, bspec, aspec, abspec],
        out_specs=[espec, wout_spec],
        out_shape=[jax.ShapeDtypeStruct((e, d), jnp.float32),
                   jax.ShapeDtypeStruct((_H, e), jnp.float32)],
    )(ep, s, w_eout, b_eout[None, :], amat, a_b[None, :])


def _scatter_body(npad, epw, bb, vals, isrc, idst, wflat, out,
                  iv1a, iv2a, wbufa, ga, sbufa,
                  iv1b, iv2b, wbufb, gb, sbufb,
                  ztb, acc, s1a, s1b, s2a, s2b, sga, sgb, sca, scb):
    d = 128
    cid = lax.axis_index("c")
    sid = lax.axis_index("s")
    wid = sid * _NC + cid
    base = wid * epw
    nb = epw // bb
    rows_per_tile = npad // _NS
    chunk = 32
    nch = rows_per_tile // chunk
    bufs = ((iv1a, iv2a, wbufa, ga, sbufa, s1a, s2a, sga, sca),
            (iv1b, iv2b, wbufb, gb, sbufb, s1b, s2b, sgb, scb))

    zeros16 = jnp.zeros((_L,), jnp.float32)
    wbufa[pl.ds(bb * _H, _L)] = zeros16
    wbufb[pl.ds(bb * _H, _L)] = zeros16

    def zrow(r, c):
        for col in range(d // _L):
            ztb[r, pl.ds(col * _L, _L)] = zeros16
        return c

    lax.fori_loop(0, chunk, zrow, 0)
    for k in range(nch):
        pltpu.sync_copy(ztb, acc.at[pl.ds(sid * rows_per_tile + k * chunk,
                                          chunk)])
    plsc.subcore_barrier()

    dnums = lax.GatherDimensionNumbers(
        offset_dims=(), collapsed_slice_dims=(0,), start_index_map=(0,))

    def fire_src_idx(j, b):
        iv1, _, _, _, _, s1, _, _, _ = bufs[b]
        pltpu.async_copy(isrc.at[pl.ds(base + j * bb, bb)], iv1, s1)

    def wait_src_idx(b):
        iv1, _, _, _, _, s1, _, _, _ = bufs[b]
        pltpu.make_async_copy(isrc.at[pl.ds(base, bb)], iv1, s1).wait()

    def fire_nidx(j, b):
        _, iv2, wbuf, _, _, _, s2, _, _ = bufs[b]
        off = base + j * bb
        pltpu.async_copy(idst.at[pl.ds(off, bb)], iv2, s2)
        pltpu.async_copy(wflat.at[pl.ds(off * _H, bb * _H)],
                         wbuf.at[pl.ds(0, bb * _H)], s2)

    def wait_nidx(b):
        _, iv2, wbuf, _, _, _, s2, _, _ = bufs[b]
        pltpu.make_async_copy(isrc.at[pl.ds(base, bb)], iv2, s2).wait()
        pltpu.make_async_copy(wflat.at[pl.ds(base, bb * _H)],
                              wbuf.at[pl.ds(0, bb * _H)], s2).wait()

    def fire_gather(b):
        _, iv2, _, g, _, _, _, sg, _ = bufs[b]
        pltpu.async_copy(vals.at[iv2], g, sg)

    def wait_gather(b):
        _, iv2, _, g, _, _, _, sg, _ = bufs[b]
        pltpu.make_async_copy(vals.at[iv2], g, sg).wait()

    def fire_scatter(b):
        iv1, _, _, _, sbuf, _, _, _, sc = bufs[b]
        pltpu.async_copy(sbuf, acc.at[iv1], sc, add=True)

    def wait_scatter(b):
        iv1, _, _, _, sbuf, _, _, _, sc = bufs[b]
        pltpu.make_async_copy(sbuf, acc.at[iv1], sc).wait()

    def compute(b):
        _, _, wbuf, g, sbuf, _, _, _, _ = bufs[b]

        def erow(e, c2):
            w16 = wbuf[pl.ds(e * _H, _L)]
            for h in range(_H):
                bc = lax.gather(
                    w16, jnp.full((_L, 1), h, jnp.int32), dnums, (1,),
                    mode=lax.GatherScatterMode.PROMISE_IN_BOUNDS)
                sl = pl.ds(h * _L, _L)
                sbuf[e, sl] = g[e, sl] * bc
            return c2

        lax.fori_loop(0, bb, erow, 0)

    fire_nidx(0, 0)
    fire_nidx(1, 1)
    wait_nidx(0)
    fire_gather(0)

    def pair(j2, carry):
        for b in range(2):
            j = j2 * 2 + b
            wait_gather(b)

            @pl.when(j2 * 2 + b >= 2)
            def _():
                wait_scatter(b)

            fire_src_idx(j, b)
            compute(b)

            @pl.when(j + 2 <= nb - 1)
            def _():
                fire_nidx(j + 2, b)

            wait_src_idx(b)
            fire_scatter(b)
            wait_nidx(1 - b)
            fire_gather(1 - b)
        return carry

    lax.fori_loop(0, (nb - 1) // 2, pair, 0)
    bl = (nb - 1) % 2
    wait_gather(bl)
    wait_scatter(bl)
    fire_src_idx(nb - 1, bl)
    compute(bl)
    wait_src_idx(bl)
    fire_scatter(bl)
    wait_scatter(1 - bl)
    wait_scatter(bl)
    plsc.subcore_barrier()
    for k in range(nch):
        r0 = sid * rows_per_tile + k * chunk
        pltpu.sync_copy(acc.at[pl.ds(r0, chunk)], ztb)
        pltpu.sync_copy(ztb, out.at[cid, pl.ds(r0, chunk)])


def _denom_body(npad, epw, bb, dpad, isrc, wflat, outd, outrep,
                iv1, wbuf, accd, pbuf, dbuf, rbuf):
    cid = lax.axis_index("c")
    sid = lax.axis_index("s")
    wid = sid * _NC + cid
    base = wid * epw
    nb = epw // bb
    rows_pt = npad // _NS
    seg = rows_pt * _H
    rch = 64

    zeros16 = jnp.zeros((_L,), jnp.float32)
    wbuf[pl.ds(bb * _H, _L)] = zeros16

    def zd(i, c):
        accd[pl.ds(i * _L, _L)] = zeros16
        return c

    lax.fori_loop(0, dpad // _L, zd, 0)
    lanemask = lax.iota(jnp.int32, _L) < _H

    def step(j, carry):
        off = base + j * bb
        pltpu.sync_copy(isrc.at[pl.ds(off, bb)], iv1)
        pltpu.sync_copy(wflat.at[pl.ds(off * _H, bb * _H)],
                        wbuf.at[pl.ds(0, bb * _H)])

        def egroup(gi, c2):
            srcs16 = iv1[pl.ds(gi * _L, _L)]
            for k in range(_L):
                e = gi * _L + k
                w16 = wbuf[pl.ds(e * _H, _L)]
                wm = jnp.where(lanemask, w16, 0.0)
                doff = srcs16[k] * _H
                accd[pl.ds(doff, _L)] = accd[pl.ds(doff, _L)] + wm
            return c2

        lax.fori_loop(0, bb // _L, egroup, 0)
        return carry

    lax.fori_loop(0, nb, step, 0)
    pltpu.sync_copy(accd, outd.at[pl.ds((cid * _NS + sid) * dpad, dpad)])
    plsc.subcore_barrier()

    def zb(i, c):
        dbuf[pl.ds(i * _L, _L)] = zeros16
        return c

    lax.fori_loop(0, seg // _L + 1, zb, 0)
    for t in range(_NS):
        pltpu.sync_copy(
            outd.at[pl.ds((cid * _NS + t) * dpad + sid * rows_pt * _H, seg)],
            pbuf.at[pl.ds(0, seg)])

        def addp(i, c):
            sl = pl.ds(i * _L, _L)
            dbuf[sl] = dbuf[sl] + pbuf[sl]
            return c

        lax.fori_loop(0, seg // _L, addp, 0)
    dnums = lax.GatherDimensionNumbers(
        offset_dims=(), collapsed_slice_dims=(0,), start_index_map=(0,))
    for ch in range(rows_pt // rch):

        def nrow(ln, c):
            w16 = dbuf[pl.ds((ch * rch + ln) * _H, _L)]
            for h in range(_H):
                rbuf[ln, pl.ds(h * _L, _L)] = jnp.full((_L,), w16[h],
                                                       jnp.float32)
            return c

        lax.fori_loop(0, rch, nrow, 0)
        pltpu.sync_copy(
            rbuf, outrep.at[cid, pl.ds(sid * rows_pt + ch * rch, rch)])


def _scatter_stage(vals, isrc, idst, w):
    n, d = vals.shape
    e = isrc.shape[0]
    epw = e // (_NC * _NS)
    bb = 80
    chunk = 32
    npad = -(-n // (_NS * chunk)) * (_NS * chunk)
    dpad = npad * _H + 128
    wflat = w.T.reshape(e * _H)
    mesh = plsc.VectorSubcoreMesh(core_axis_name="c", subcore_axis_name="s", num_cores=_NC, num_subcores=_NS)
    kern = pl.kernel(
        functools.partial(_scatter_body, npad, epw, bb),
        out_type=jax.ShapeDtypeStruct((_NC, npad, d), jnp.float32),
        mesh=mesh,
        scratch_types=[
            pltpu.VMEM((bb,), jnp.int32),
            pltpu.VMEM((bb,), jnp.int32),
            pltpu.VMEM((bb * _H + _L,), jnp.float32),
            pltpu.VMEM((bb, d), jnp.float32),
            pltpu.VMEM((bb, d), jnp.float32),
            pltpu.VMEM((bb,), jnp.int32),
            pltpu.VMEM((bb,), jnp.int32),
            pltpu.VMEM((bb * _H + _L,), jnp.float32),
            pltpu.VMEM((bb, d), jnp.float32),
            pltpu.VMEM((bb, d), jnp.float32),
            pltpu.VMEM((chunk, d), jnp.float32),
            pltpu.VMEM_SHARED((npad, d), jnp.float32),
            pltpu.SemaphoreType.DMA,
            pltpu.SemaphoreType.DMA,
            pltpu.SemaphoreType.DMA,
            pltpu.SemaphoreType.DMA,
            pltpu.SemaphoreType.DMA,
            pltpu.SemaphoreType.DMA,
            pltpu.SemaphoreType.DMA,
            pltpu.SemaphoreType.DMA,
        ],
    )
    acc = kern(vals, isrc, idst, wflat)
    dbb = 2000
    rows_pt = npad // _NS
    seg = rows_pt * _H + _L
    dkern = pl.kernel(
        functools.partial(_denom_body, npad, epw, dbb, dpad),
        out_type=[jax.ShapeDtypeStruct((_NC * _NS * dpad,), jnp.float32),
                  jax.ShapeDtypeStruct((_NC, npad, d), jnp.float32)],
        mesh=mesh,
        scratch_types=[
            pltpu.VMEM((dbb,), jnp.int32),
            pltpu.VMEM((dbb * _H + _L,), jnp.float32),
            pltpu.VMEM((dpad,), jnp.float32),
            pltpu.VMEM((seg,), jnp.float32),
            pltpu.VMEM((seg,), jnp.float32),
            pltpu.VMEM((64, d), jnp.float32),
        ],
    )
    _, rep = dkern(isrc, wflat)
    return acc, rep


def _final_body(acc_ref, rep_ref, nn_ref):
    num = acc_ref[0] + acc_ref[1]
    den = rep_ref[0] + rep_ref[1]
    nn_ref[...] = num / jnp.where(den == 0.0, 1.0, den)


def _finalize(acc, rep, n, block=1000):
    d = 128
    grid = n // block
    spec = pl.BlockSpec((_NC, block, d), lambda i: (0, i, 0))
    return pl.pallas_call(
        _final_body,
        grid=(grid,),
        in_specs=[spec, spec],
        out_specs=pl.BlockSpec((block, d), lambda i: (i, 0)),
        out_shape=jax.ShapeDtypeStruct((n, d), jnp.float32),
    )(acc, rep)


def kernel(nodes, edges, edge_index, spatial_edge_encoding,
           W_src, b_src, W_dst, b_dst, W_edge, b_edge,
           a_w, a_b, W_nout, b_nout, W_eout, b_eout):
    isrc = edge_index[0]
    idst = edge_index[1]
    x_src, x_dst, vals = _node_proj(nodes, W_src, b_src, W_dst, b_dst,
                                    W_nout, b_nout)
    ep = _edge_pre(edges, spatial_edge_encoding, W_edge, b_edge)
    s = _edge_gather_sum(x_src, x_dst, isrc, idst)
    new_edges, w = _edge_stage(ep, s, W_eout, b_eout, a_w, a_b)
    acc, rep = _scatter_stage(vals, isrc, idst, w)
    new_nodes = _finalize(acc, rep, nodes.shape[0])
    return new_nodes, new_edges


# speedup vs baseline: 1.0906x; 1.0906x over previous
import functools

import jax
import jax.numpy as jnp
from jax import lax
from jax.experimental import pallas as pl
from jax.experimental.pallas import tpu as pltpu
from jax.experimental.pallas import tpu_sc as plsc

_H = 8
_DH = 16
_NC = 2
_NS = 16
_L = 16


def _proj_body(nodes_ref, wsrc_ref, bsrc_ref, wdst_ref, bdst_ref,
               wval_ref, bval_ref, xsrc_ref, xdst_ref, vals_ref):
    x = nodes_ref[...]
    xsrc_ref[...] = jnp.dot(x, wsrc_ref[...],
                            preferred_element_type=jnp.float32) + bsrc_ref[...]
    xdst_ref[...] = jnp.dot(x, wdst_ref[...],
                            preferred_element_type=jnp.float32) + bdst_ref[...]
    vals_ref[...] = jnp.dot(x, wval_ref[...],
                            preferred_element_type=jnp.float32) + bval_ref[...]


def _node_proj(nodes, w_src, b_src, w_dst, b_dst, w_val, b_val, block=1000):
    n, d = nodes.shape
    grid = n // block
    wspec = pl.BlockSpec((d, d), lambda i: (0, 0))
    bspec = pl.BlockSpec((1, d), lambda i: (0, 0))
    xspec = pl.BlockSpec((block, d), lambda i: (i, 0))
    out = jax.ShapeDtypeStruct((n, d), jnp.float32)
    return pl.pallas_call(
        _proj_body,
        grid=(grid,),
        in_specs=[xspec, wspec, bspec, wspec, bspec, wspec, bspec],
        out_specs=[xspec, xspec, xspec],
        out_shape=[out, out, out],
    )(nodes, w_src, b_src[None, :], w_dst, b_dst[None, :], w_val, b_val[None, :])


def _gather_body(epw, bb, psrc, pdst, isrc, idst, s_out,
                 iv1a, iv2a, g1a, g2a, iv1b, iv2b, g1b, g2b,
                 sia, sib, sga, sgb, ssa, ssb):
    cid = lax.axis_index("c")
    sid = lax.axis_index("s")
    wid = sid * _NC + cid
    base = wid * epw
    nb = epw // bb
    bufs = ((iv1a, iv2a, g1a, g2a, sia, sga, ssa),
            (iv1b, iv2b, g1b, g2b, sib, sgb, ssb))

    def fire_idx(j, b):
        iv1, iv2, _, _, si, _, _ = bufs[b]
        off = base + j * bb
        pltpu.async_copy(isrc.at[pl.ds(off, bb)], iv1, si)
        pltpu.async_copy(idst.at[pl.ds(off, bb)], iv2, si)

    def wait_idx(b):
        iv1, iv2, _, _, si, _, _ = bufs[b]
        pltpu.make_async_copy(isrc.at[pl.ds(base, bb)], iv1, si).wait()
        pltpu.make_async_copy(isrc.at[pl.ds(base, bb)], iv2, si).wait()

    def fire_gather(b):
        iv1, iv2, g1, g2, _, sg, _ = bufs[b]
        pltpu.async_copy(psrc.at[iv1], g1, sg)
        pltpu.async_copy(pdst.at[iv2], g2, sg)

    def wait_gather(b):
        iv1, iv2, g1, g2, _, sg, _ = bufs[b]
        pltpu.make_async_copy(psrc.at[iv1], g1, sg).wait()
        pltpu.make_async_copy(pdst.at[iv2], g2, sg).wait()

    def fire_store(j, b):
        _, _, g1, _, _, _, ss = bufs[b]
        off = base + j * bb
        pltpu.async_copy(g1, s_out.at[pl.ds(off, bb)], ss)

    def wait_store(b):
        _, _, g1, _, _, _, ss = bufs[b]
        pltpu.make_async_copy(g1, s_out.at[pl.ds(base, bb)], ss).wait()

    def compute(b):
        _, _, g1, g2, _, _, _ = bufs[b]

        def erow(e, c2):
            for h in range(_H):
                sl = pl.ds(h * _L, _L)
                g1[e, sl] = g1[e, sl] + g2[e, sl]
            return c2

        lax.fori_loop(0, bb, erow, 0)

    fire_idx(0, 0)
    fire_idx(1, 1)
    wait_idx(0)
    fire_gather(0)

    def pair(j2, carry):
        for b in range(2):
            j = j2 * 2 + b
            wait_gather(b)
            compute(b)
            fire_store(j, b)
            wait_idx(1 - b)

            @pl.when(j2 * 2 + b >= 1)
            def _():
                wait_store(1 - b)

            fire_gather(1 - b)
            @pl.when(j + 2 <= nb - 1)
            def _():
                fire_idx(j + 2, b)
        return carry

    lax.fori_loop(0, (nb - 1) // 2, pair, 0)
    bl = (nb - 1) % 2
    wait_gather(bl)
    compute(bl)
    fire_store(nb - 1, bl)
    wait_store(1 - bl)
    wait_store(bl)


def _edge_gather_sum(x_src, x_dst, isrc, idst):
    n, d = x_src.shape
    e = isrc.shape[0]
    epw = e // (_NC * _NS)
    bb = 80
    mesh = plsc.VectorSubcoreMesh(core_axis_name="c", subcore_axis_name="s", num_cores=_NC, num_subcores=_NS)
    kern = pl.kernel(
        functools.partial(_gather_body, epw, bb),
        out_type=jax.ShapeDtypeStruct((e, d), jnp.float32),
        mesh=mesh,
        scratch_types=[
            pltpu.VMEM((bb,), jnp.int32),
            pltpu.VMEM((bb,), jnp.int32),
            pltpu.VMEM((bb, d), jnp.float32),
            pltpu.VMEM((bb, d), jnp.float32),
            pltpu.VMEM((bb,), jnp.int32),
            pltpu.VMEM((bb,), jnp.int32),
            pltpu.VMEM((bb, d), jnp.float32),
            pltpu.VMEM((bb, d), jnp.float32),
            pltpu.SemaphoreType.DMA,
            pltpu.SemaphoreType.DMA,
            pltpu.SemaphoreType.DMA,
            pltpu.SemaphoreType.DMA,
            pltpu.SemaphoreType.DMA,
            pltpu.SemaphoreType.DMA,
        ],
    )
    return kern(x_src, x_dst, isrc, idst)


def _edge_body(edges_ref, spe_ref, s_ref, wedge_ref, bedge_ref,
               weout_ref, beout_ref, amat_ref, ab_ref, ne_ref, w_ref):
    z = jnp.dot(edges_ref[...], wedge_ref[...],
                preferred_element_type=jnp.float32)
    z = z + bedge_ref[...] + s_ref[...] + spe_ref[...]
    t = 0.5 * z * (1.0 + lax.erf(z * 0.7071067811865476))
    ne_ref[...] = jnp.dot(t, weout_ref[...],
                          preferred_element_type=jnp.float32) + beout_ref[...]
    logits = jnp.dot(t, amat_ref[...],
                     preferred_element_type=jnp.float32) + ab_ref[...]
    w_ref[...] = jnp.exp(logits).T


def _edge_stage(edges, spe, s, w_edge, b_edge, w_eout, b_eout, a_w, a_b,
                block=512):
    e, d = edges.shape
    grid = e // block
    amat = jnp.zeros((d, _H), jnp.float32)
    rows = jnp.arange(d)
    amat = amat.at[rows, rows // _DH].set(a_w.reshape(d))
    espec = pl.BlockSpec((block, d), lambda i: (i, 0))
    wspec = pl.BlockSpec((d, d), lambda i: (0, 0))
    bspec = pl.BlockSpec((1, d), lambda i: (0, 0))
    aspec = pl.BlockSpec((d, _H), lambda i: (0, 0))
    abspec = pl.BlockSpec((1, _H), lambda i: (0, 0))
    wout_spec = pl.BlockSpec((_H, block), lambda i: (0, i))
    return pl.pallas_call(
        _edge_body,
        grid=(grid,),
        in_specs=[espec, espec, espec, wspec, bspec, wspec, bspec, aspec,
                  abspec],
        out_specs=[espec, wout_spec],
        out_shape=[jax.ShapeDtypeStruct((e, d), jnp.float32),
                   jax.ShapeDtypeStruct((_H, e), jnp.float32)],
    )(edges, spe, s, w_edge, b_edge[None, :], w_eout, b_eout[None, :],
      amat, a_b[None, :])


def _scatter_body(npad, epw, bb, vals, isrc, idst, wflat, out,
                  iv1a, iv2a, wbufa, ga, sbufa,
                  iv1b, iv2b, wbufb, gb, sbufb,
                  ztb, acc, s1a, s1b, s2a, s2b, sga, sgb, sca, scb):
    d = 128
    cid = lax.axis_index("c")
    sid = lax.axis_index("s")
    wid = sid * _NC + cid
    base = wid * epw
    nb = epw // bb
    rows_per_tile = npad // _NS
    chunk = 32
    nch = rows_per_tile // chunk
    bufs = ((iv1a, iv2a, wbufa, ga, sbufa, s1a, s2a, sga, sca),
            (iv1b, iv2b, wbufb, gb, sbufb, s1b, s2b, sgb, scb))

    zeros16 = jnp.zeros((_L,), jnp.float32)
    wbufa[pl.ds(bb * _H, _L)] = zeros16
    wbufb[pl.ds(bb * _H, _L)] = zeros16

    def zrow(r, c):
        for col in range(d // _L):
            ztb[r, pl.ds(col * _L, _L)] = zeros16
        return c

    lax.fori_loop(0, chunk, zrow, 0)
    for k in range(nch):
        pltpu.sync_copy(ztb, acc.at[pl.ds(sid * rows_per_tile + k * chunk,
                                          chunk)])
    plsc.subcore_barrier()

    dnums = lax.GatherDimensionNumbers(
        offset_dims=(), collapsed_slice_dims=(0,), start_index_map=(0,))

    def fire_src_idx(j, b):
        iv1, _, _, _, _, s1, _, _, _ = bufs[b]
        pltpu.async_copy(isrc.at[pl.ds(base + j * bb, bb)], iv1, s1)

    def wait_src_idx(b):
        iv1, _, _, _, _, s1, _, _, _ = bufs[b]
        pltpu.make_async_copy(isrc.at[pl.ds(base, bb)], iv1, s1).wait()

    def fire_nidx(j, b):
        _, iv2, wbuf, _, _, _, s2, _, _ = bufs[b]
        off = base + j * bb
        pltpu.async_copy(idst.at[pl.ds(off, bb)], iv2, s2)
        pltpu.async_copy(wflat.at[pl.ds(off * _H, bb * _H)],
                         wbuf.at[pl.ds(0, bb * _H)], s2)

    def wait_nidx(b):
        _, iv2, wbuf, _, _, _, s2, _, _ = bufs[b]
        pltpu.make_async_copy(isrc.at[pl.ds(base, bb)], iv2, s2).wait()
        pltpu.make_async_copy(wflat.at[pl.ds(base, bb * _H)],
                              wbuf.at[pl.ds(0, bb * _H)], s2).wait()

    def fire_gather(b):
        _, iv2, _, g, _, _, _, sg, _ = bufs[b]
        pltpu.async_copy(vals.at[iv2], g, sg)

    def wait_gather(b):
        _, iv2, _, g, _, _, _, sg, _ = bufs[b]
        pltpu.make_async_copy(vals.at[iv2], g, sg).wait()

    def fire_scatter(b):
        iv1, _, _, _, sbuf, _, _, _, sc = bufs[b]
        pltpu.async_copy(sbuf, acc.at[iv1], sc, add=True)

    def wait_scatter(b):
        iv1, _, _, _, sbuf, _, _, _, sc = bufs[b]
        pltpu.make_async_copy(sbuf, acc.at[iv1], sc).wait()

    def compute(b):
        _, _, wbuf, g, sbuf, _, _, _, _ = bufs[b]

        def erow(e, c2):
            w16 = wbuf[pl.ds(e * _H, _L)]
            for h in range(_H):
                bc = lax.gather(
                    w16, jnp.full((_L, 1), h, jnp.int32), dnums, (1,),
                    mode=lax.GatherScatterMode.PROMISE_IN_BOUNDS)
                sl = pl.ds(h * _L, _L)
                sbuf[e, sl] = g[e, sl] * bc
            return c2

        lax.fori_loop(0, bb, erow, 0)

    fire_nidx(0, 0)
    fire_nidx(1, 1)
    wait_nidx(0)
    fire_gather(0)

    def pair(j2, carry):
        for b in range(2):
            j = j2 * 2 + b
            wait_gather(b)

            @pl.when(j2 * 2 + b >= 2)
            def _():
                wait_scatter(b)

            fire_src_idx(j, b)
            compute(b)

            @pl.when(j + 2 <= nb - 1)
            def _():
                fire_nidx(j + 2, b)

            wait_src_idx(b)
            fire_scatter(b)
            wait_nidx(1 - b)
            fire_gather(1 - b)
        return carry

    lax.fori_loop(0, (nb - 1) // 2, pair, 0)
    bl = (nb - 1) % 2
    wait_gather(bl)
    wait_scatter(bl)
    fire_src_idx(nb - 1, bl)
    compute(bl)
    wait_src_idx(bl)
    fire_scatter(bl)
    wait_scatter(1 - bl)
    wait_scatter(bl)
    plsc.subcore_barrier()
    for k in range(nch):
        r0 = sid * rows_per_tile + k * chunk
        pltpu.sync_copy(acc.at[pl.ds(r0, chunk)], ztb)
        pltpu.sync_copy(ztb, out.at[cid, pl.ds(r0, chunk)])


def _denom_body(npad, epw, bb, dpad, isrc, wflat, outd, outrep,
                iv1, wbuf, accd, pbuf, dbuf, rbuf):
    cid = lax.axis_index("c")
    sid = lax.axis_index("s")
    wid = sid * _NC + cid
    base = wid * epw
    nb = epw // bb
    rows_pt = npad // _NS
    seg = rows_pt * _H
    rch = 64

    zeros16 = jnp.zeros((_L,), jnp.float32)
    wbuf[pl.ds(bb * _H, _L)] = zeros16

    def zd(i, c):
        accd[pl.ds(i * _L, _L)] = zeros16
        return c

    lax.fori_loop(0, dpad // _L, zd, 0)
    lanemask = lax.iota(jnp.int32, _L) < _H

    def step(j, carry):
        off = base + j * bb
        pltpu.sync_copy(isrc.at[pl.ds(off, bb)], iv1)
        pltpu.sync_copy(wflat.at[pl.ds(off * _H, bb * _H)],
                        wbuf.at[pl.ds(0, bb * _H)])

        def egroup(gi, c2):
            srcs16 = iv1[pl.ds(gi * _L, _L)]
            for k in range(_L):
                e = gi * _L + k
                w16 = wbuf[pl.ds(e * _H, _L)]
                wm = jnp.where(lanemask, w16, 0.0)
                doff = srcs16[k] * _H
                accd[pl.ds(doff, _L)] = accd[pl.ds(doff, _L)] + wm
            return c2

        lax.fori_loop(0, bb // _L, egroup, 0)
        return carry

    lax.fori_loop(0, nb, step, 0)
    pltpu.sync_copy(accd, outd.at[pl.ds((cid * _NS + sid) * dpad, dpad)])
    plsc.subcore_barrier()

    def zb(i, c):
        dbuf[pl.ds(i * _L, _L)] = zeros16
        return c

    lax.fori_loop(0, seg // _L + 1, zb, 0)
    for t in range(_NS):
        pltpu.sync_copy(
            outd.at[pl.ds((cid * _NS + t) * dpad + sid * rows_pt * _H, seg)],
            pbuf.at[pl.ds(0, seg)])

        def addp(i, c):
            sl = pl.ds(i * _L, _L)
            dbuf[sl] = dbuf[sl] + pbuf[sl]
            return c

        lax.fori_loop(0, seg // _L, addp, 0)
    dnums = lax.GatherDimensionNumbers(
        offset_dims=(), collapsed_slice_dims=(0,), start_index_map=(0,))
    for ch in range(rows_pt // rch):

        def nrow(ln, c):
            w16 = dbuf[pl.ds((ch * rch + ln) * _H, _L)]
            for h in range(_H):
                rbuf[ln, pl.ds(h * _L, _L)] = jnp.full((_L,), w16[h],
                                                       jnp.float32)
            return c

        lax.fori_loop(0, rch, nrow, 0)
        pltpu.sync_copy(
            rbuf, outrep.at[cid, pl.ds(sid * rows_pt + ch * rch, rch)])


def _scatter_stage(vals, isrc, idst, w):
    n, d = vals.shape
    e = isrc.shape[0]
    epw = e // (_NC * _NS)
    bb = 80
    chunk = 32
    npad = -(-n // (_NS * chunk)) * (_NS * chunk)
    dpad = npad * _H + 128
    wflat = w.T.reshape(e * _H)
    mesh = plsc.VectorSubcoreMesh(core_axis_name="c", subcore_axis_name="s", num_cores=_NC, num_subcores=_NS)
    kern = pl.kernel(
        functools.partial(_scatter_body, npad, epw, bb),
        out_type=jax.ShapeDtypeStruct((_NC, npad, d), jnp.float32),
        mesh=mesh,
        scratch_types=[
            pltpu.VMEM((bb,), jnp.int32),
            pltpu.VMEM((bb,), jnp.int32),
            pltpu.VMEM((bb * _H + _L,), jnp.float32),
            pltpu.VMEM((bb, d), jnp.float32),
            pltpu.VMEM((bb, d), jnp.float32),
            pltpu.VMEM((bb,), jnp.int32),
            pltpu.VMEM((bb,), jnp.int32),
            pltpu.VMEM((bb * _H + _L,), jnp.float32),
            pltpu.VMEM((bb, d), jnp.float32),
            pltpu.VMEM((bb, d), jnp.float32),
            pltpu.VMEM((chunk, d), jnp.float32),
            pltpu.VMEM_SHARED((npad, d), jnp.float32),
            pltpu.SemaphoreType.DMA,
            pltpu.SemaphoreType.DMA,
            pltpu.SemaphoreType.DMA,
            pltpu.SemaphoreType.DMA,
            pltpu.SemaphoreType.DMA,
            pltpu.SemaphoreType.DMA,
            pltpu.SemaphoreType.DMA,
            pltpu.SemaphoreType.DMA,
        ],
    )
    acc = kern(vals, isrc, idst, wflat)
    dbb = 2000
    rows_pt = npad // _NS
    seg = rows_pt * _H + _L
    dkern = pl.kernel(
        functools.partial(_denom_body, npad, epw, dbb, dpad),
        out_type=[jax.ShapeDtypeStruct((_NC * _NS * dpad,), jnp.float32),
                  jax.ShapeDtypeStruct((_NC, npad, d), jnp.float32)],
        mesh=mesh,
        scratch_types=[
            pltpu.VMEM((dbb,), jnp.int32),
            pltpu.VMEM((dbb * _H + _L,), jnp.float32),
            pltpu.VMEM((dpad,), jnp.float32),
            pltpu.VMEM((seg,), jnp.float32),
            pltpu.VMEM((seg,), jnp.float32),
            pltpu.VMEM((64, d), jnp.float32),
        ],
    )
    _, rep = dkern(isrc, wflat)
    return acc, rep


def _final_body(acc_ref, rep_ref, nn_ref):
    num = acc_ref[0] + acc_ref[1]
    den = rep_ref[0] + rep_ref[1]
    nn_ref[...] = num / jnp.where(den == 0.0, 1.0, den)


def _finalize(acc, rep, n, block=1000):
    d = 128
    grid = n // block
    spec = pl.BlockSpec((_NC, block, d), lambda i: (0, i, 0))
    return pl.pallas_call(
        _final_body,
        grid=(grid,),
        in_specs=[spec, spec],
        out_specs=pl.BlockSpec((block, d), lambda i: (i, 0)),
        out_shape=jax.ShapeDtypeStruct((n, d), jnp.float32),
    )(acc, rep)


def kernel(nodes, edges, edge_index, spatial_edge_encoding,
           W_src, b_src, W_dst, b_dst, W_edge, b_edge,
           a_w, a_b, W_nout, b_nout, W_eout, b_eout):
    isrc = edge_index[0]
    idst = edge_index[1]
    x_src, x_dst, vals = _node_proj(nodes, W_src, b_src, W_dst, b_dst,
                                    W_nout, b_nout)
    s = _edge_gather_sum(x_src, x_dst, isrc, idst)
    new_edges, w = _edge_stage(edges, spatial_edge_encoding, s,
                               W_edge, b_edge, W_eout, b_eout, a_w, a_b)
    acc, rep = _scatter_stage(vals, isrc, idst, w)
    new_nodes = _finalize(acc, rep, nodes.shape[0])
    return new_nodes, new_edges
